# baseline (speedup 1.0000x reference)
import functools

import jax
import jax.numpy as jnp
from jax import lax
from jax.experimental import pallas as pl
from jax.experimental.pallas import tpu as pltpu
from jax.experimental.pallas import tpu_sc as plsc

B = 4096
L = 50
F = 26
D = 32
GR = 4
QS = 64

_info = plsc.get_sparse_core_info()
NC, NS = _info.num_cores, _info.num_subcores
NW = NC * NS
RW = B // NW
NB = 4
NBL = NB * L
N_SB = RW // NB
NCH = 2
CH = RW // NCH
NCHUNK = F * NCH

_mesh = plsc.VectorSubcoreMesh(core_axis_name="c", subcore_axis_name="s")


@functools.partial(
    pl.kernel,
    mesh=_mesh,
    out_type=(
        jax.ShapeDtypeStruct((B, D), jnp.float32),
        jax.ShapeDtypeStruct((F, B, D), jnp.float32),
    ),
    scratch_types=[
        pltpu.VMEM((RW * L,), jnp.int32),
        pltpu.VMEM((RW * QS,), jnp.int32),
        pltpu.VMEM((F, RW), jnp.int32),
        pltpu.VMEM((F, RW), jnp.int32),
        pltpu.VMEM((NBL, GR * D), jnp.float32),
        pltpu.VMEM((NBL, GR * D), jnp.float32),
        pltpu.VMEM((8, D), jnp.float32),
        pltpu.VMEM((8, D), jnp.float32),
        pltpu.VMEM((CH, GR * D), jnp.float32),
        pltpu.VMEM((CH, GR * D), jnp.float32),
        pltpu.VMEM((CH, D), jnp.float32),
        pltpu.VMEM((CH, D), jnp.float32),
        pltpu.SemaphoreType.DMA,
        pltpu.SemaphoreType.DMA,
        pltpu.SemaphoreType.DMA,
        pltpu.SemaphoreType.DMA,
        pltpu.SemaphoreType.DMA,
        pltpu.SemaphoreType.DMA,
    ],
)
def _sc_kernel(item_r, item_q, ctx_r, ctx_q, item_t, ctx_t,
               seq_out, ctx_out,
               ir_v, iq_v, cr_v, cq_v, ig0, ig1, sb0, sb1,
               cg0, cg1, cp0, cp1,
               isem, icsem, gsem, cgsem, ssem, csem):
    wid = lax.axis_index("s") * NC + lax.axis_index("c")
    base = wid * RW

    cp_ir = pltpu.async_copy(item_r.at[pl.ds(base * L, RW * L)], ir_v, isem)
    cp_iq = pltpu.async_copy(item_q.at[pl.ds(base * QS, RW * QS)], iq_v, isem)
    cp_cr = pltpu.async_copy(ctx_r.at[:, pl.ds(base, RW)], cr_v, icsem)
    cp_cq = pltpu.async_copy(ctx_q.at[:, pl.ds(base, RW)], cq_v, icsem)
    cp_cr.wait()
    cp_cq.wait()

    cgs = [cg0, cg1]
    cps = [cp0, cp1]

    def ctx_issue(c_val, buf):
        f = c_val // NCH
        hc = c_val % NCH
        pltpu.async_copy(
            ctx_t.at[cr_v.at[f, pl.ds(hc * CH, CH)]], buf, cgsem)

    def ctx_wait_gather(buf):
        pltpu.make_async_copy(ctx_t.at[pl.ds(0, CH)], buf, cgsem).wait()

    def ctx_wait_store(buf):
        pltpu.make_async_copy(
            buf, ctx_out.at[0, pl.ds(base, CH), :], csem).wait()

    def ctx_repack(c_val, grp, pkd):
        f = c_val // NCH
        hc = c_val % NCH

        def body(g, carry):
            qv = cq_v[f, pl.ds(hc * CH + g * 16, 16)]
            for k in range(16):
                j = g * 16 + k
                q = qv[k]
                pkd[j, pl.ds(0, 16)] = grp[j, pl.ds(q, 16)]
                pkd[j, pl.ds(16, 16)] = grp[j, pl.ds(q + 16, 16)]
            return carry

        lax.fori_loop(0, CH // 16, body, 0)

    def ctx_store(c_val, pkd):
        f = c_val // NCH
        hc = c_val % NCH
        pltpu.async_copy(
            pkd, ctx_out.at[f, pl.ds(base + hc * CH, CH), :], csem)

    ctx_issue(0, cg0)
    ctx_issue(1, cg1)

    def ctx_outer(cc, carry):
        for par in range(2):
            c = cc * 2 + par
            pl.when(c >= 2)(lambda: ctx_wait_store(cps[par]))
            ctx_wait_gather(cgs[par])
            ctx_repack(c, cgs[par], cps[par])
            pl.when(c + 2 < NCHUNK)(
                functools.partial(ctx_issue, c + 2, cgs[par]))
            ctx_store(c, cps[par])
        return carry

    lax.fori_loop(0, NCHUNK // 2, ctx_outer, 0)

    cp_ir.wait()
    cp_iq.wait()
    igs = [ig0, ig1]
    sbufs = [sb0, sb1]

    def issue_gather(sb_val, buf):
        pltpu.async_copy(
            item_t.at[ir_v.at[pl.ds(sb_val * NBL, NBL)]], buf, gsem)

    def wait_gather(buf):
        pltpu.make_async_copy(item_t.at[pl.ds(0, NBL)], buf, gsem).wait()

    def wait_store(buf):
        pltpu.make_async_copy(
            buf, seq_out.at[pl.ds(base, 8), :], ssem).wait()

    def reduce_sub(rows, sb_val, sbuf, quarter):
        def body(b, carry):
            rloc = sb_val * NB + b
            acc0 = jnp.zeros((16,), jnp.float32)
            acc1 = jnp.zeros((16,), jnp.float32)
            for lg in range(4):
                qv = iq_v[pl.ds(rloc * QS + lg * 16, 16)]
                for k in range(16):
                    l = lg * 16 + k
                    if l >= L:
                        break
                    q = qv[k]
                    acc0 = acc0 + rows[b * L + l, pl.ds(q, 16)]
                    acc1 = acc1 + rows[b * L + l, pl.ds(q + 16, 16)]
            sbuf[quarter * NB + b, pl.ds(0, 16)] = acc0 * (1.0 / L)
            sbuf[quarter * NB + b, pl.ds(16, 16)] = acc1 * (1.0 / L)
            return carry

        lax.fori_loop(0, NB, body, 0)

    issue_gather(0, ig0)
    issue_gather(1, ig1)

    SB_PER_ST = 8 // NB

    def outer_body(u, carry):
        for h in range(2):
            g8 = u * 2 + h
            pl.when(g8 >= 2)(lambda: wait_store(sbufs[h]))
            for j in range(SB_PER_ST):
                sb = g8 * SB_PER_ST + j
                buf = igs[j % 2]
                wait_gather(buf)
                reduce_sub(buf, sb, sbufs[h], j)
                pl.when(sb + 2 < N_SB)(
                    functools.partial(issue_gather, sb + 2, buf))
            pltpu.async_copy(
                sbufs[h],
                seq_out.at[pl.ds(base + g8 * 8, 8), :],
                ssem,
            )
        return carry

    lax.fori_loop(0, N_SB // (2 * SB_PER_ST), outer_body, 0)

    ctx_wait_store(cp0)
    ctx_wait_store(cp1)
    wait_store(sb0)
    wait_store(sb1)


def _group_view(table):
    v = table.shape[0]
    return table.reshape(v // 32, 4, 8, D).transpose(0, 2, 1, 3).reshape(
        v // GR, GR * D)


def kernel(item_seq, context_ids, item_table, context_table):
    item_r = (8 * (item_seq // 32) + item_seq % 8).reshape(-1)
    item_q = jnp.pad(((item_seq % 32) // 8) * D,
                     ((0, 0), (0, QS - L))).reshape(-1)
    ctx_r = (8 * (context_ids // 32) + context_ids % 8).T
    ctx_q = (((context_ids % 32) // 8) * D).T
    item_t = _group_view(item_table)
    ctx_t = _group_view(context_table)
    seq_out, ctx_out = _sc_kernel(item_r, item_q, ctx_r, ctx_q, item_t, ctx_t)
    return jnp.concatenate(
        [seq_out, ctx_out.transpose(1, 0, 2).reshape(B, F * D)], axis=-1)


# speedup vs baseline: 1.5019x; 1.5019x over previous
import functools

import jax
import jax.numpy as jnp
from jax import lax
from jax.experimental import pallas as pl
from jax.experimental.pallas import tpu as pltpu
from jax.experimental.pallas import tpu_sc as plsc

B = 4096
L = 50
F = 26
D = 32

_info = plsc.get_sparse_core_info()
NC, NS = _info.num_cores, _info.num_subcores
NW = NC * NS
RW = B // NW
NB = 16
N_SB = RW // NB
CBUF = 4

_mesh = plsc.VectorSubcoreMesh(core_axis_name="c", subcore_axis_name="s")


@functools.partial(
    pl.kernel,
    mesh=_mesh,
    out_type=jax.ShapeDtypeStruct((B, D + F * D), jnp.float32),
    scratch_types=[
        pltpu.VMEM((RW * L,), jnp.int32),
        pltpu.VMEM((F, RW), jnp.int32),
        pltpu.VMEM((NB * L, D), jnp.float32),
        pltpu.VMEM((NB * L, D), jnp.float32),
        pltpu.VMEM((NB, D), jnp.float32),
        pltpu.VMEM((NB, D), jnp.float32),
        pltpu.VMEM((RW, D), jnp.float32),
        pltpu.VMEM((RW, D), jnp.float32),
        pltpu.VMEM((RW, D), jnp.float32),
        pltpu.VMEM((RW, D), jnp.float32),
        pltpu.SemaphoreType.DMA,
        pltpu.SemaphoreType.DMA,
        pltpu.SemaphoreType.DMA,
        pltpu.SemaphoreType.DMA,
        pltpu.SemaphoreType.DMA,
        pltpu.SemaphoreType.DMA,
    ],
    compiler_params=pltpu.CompilerParams(use_tc_tiling_on_sc=False),
)
def _sc_kernel(seq_flat, ctx_t, item_table, ctx_table, out,
               item_idx_v, ctx_idx_v, ir0, ir1, sb0, sb1,
               cb0, cb1, cb2, cb3,
               isem, icsem, gsem, cgsem, ssem, csem):
    wid = lax.axis_index("s") * NC + lax.axis_index("c")
    base = wid * RW

    cp_i = pltpu.async_copy(seq_flat.at[pl.ds(base * L, RW * L)], item_idx_v, isem)
    cp_c = pltpu.async_copy(ctx_t.at[:, pl.ds(base, RW)], ctx_idx_v, icsem)
    cp_c.wait()

    cbufs = [cb0, cb1, cb2, cb3]
    cg = [None] * F
    cs = [None] * F

    def ctx_store(f):
        return pltpu.async_copy(
            cbufs[f % CBUF],
            out.at[pl.ds(base, RW), pl.ds(D + f * D, D)],
            csem,
        )

    for f in range(F):
        if f >= CBUF:
            cs[f - CBUF].wait()
        cg[f] = pltpu.async_copy(
            ctx_table.at[ctx_idx_v.at[f]], cbufs[f % CBUF], cgsem)
        if f >= 1:
            cg[f - 1].wait()
            cs[f - 1] = ctx_store(f - 1)
    cg[F - 1].wait()
    cs[F - 1] = ctx_store(F - 1)

    cp_i.wait()
    irows = [ir0, ir1]
    sbufs = [sb0, sb1]
    ig = [None] * N_SB
    st = [None] * N_SB

    def reduce_block(rows_ref, sbuf_ref):
        def body(b, carry):
            acc0 = jnp.zeros((16,), jnp.float32)
            acc1 = jnp.zeros((16,), jnp.float32)
            r0 = b * L
            for l in range(L):
                acc0 = acc0 + rows_ref[r0 + l, pl.ds(0, 16)]
                acc1 = acc1 + rows_ref[r0 + l, pl.ds(16, 16)]
            sbuf_ref[b, pl.ds(0, 16)] = acc0 * (1.0 / L)
            sbuf_ref[b, pl.ds(16, 16)] = acc1 * (1.0 / L)
            return carry

        lax.fori_loop(0, NB, body, 0)

    def item_step(p):
        if p >= 2:
            st[p - 2].wait()
        ig[p].wait()
        reduce_block(irows[p % 2], sbufs[p % 2])
        st[p] = pltpu.async_copy(
            sbufs[p % 2],
            out.at[pl.ds(base + p * NB, NB), pl.ds(0, D)],
            ssem,
        )

    for sb in range(N_SB):
        ig[sb] = pltpu.async_copy(
            item_table.at[item_idx_v.at[pl.ds(sb * NB * L, NB * L)]],
            irows[sb % 2], gsem)
        if sb >= 1:
            item_step(sb - 1)
    item_step(N_SB - 1)

    for f in range(F - CBUF, F):
        cs[f].wait()
    st[N_SB - 2].wait()
    st[N_SB - 1].wait()


def kernel(item_seq, context_ids, item_table, context_table):
    seq_flat = item_seq.reshape(-1)
    ctx_t = context_ids.T
    return _sc_kernel(seq_flat, ctx_t, item_table, context_table)
